# baseline (speedup 1.0000x reference)
import functools

import jax
import jax.numpy as jnp
from jax import lax
from jax.experimental import pallas as pl
from jax.experimental.pallas import tpu as pltpu
from jax.experimental.pallas import tpu_sc as plsc

N = 10000
D = 128
E = 320000

NC = 2
NS = 16
HD = D // NC
CHUNK = 128
NBUF = 1
NCHUNKS = 160
E_PAD_T = NCHUNKS * CHUNK
N_PAD = 10016
ROWS_PER_TILE = N_PAD // NS
CW = 8


def _sc_body(with_counts, x2_hbm, srcs_hbm, dsts_hbm, zrow_hbm, zcnt_hbm,
             ones_hbm, ssum_hbm, cnt_hbm, src_v, dst_v,
             bufs, ones_v, acc_s, cacc_s, gsems):
    c = lax.axis_index("c")
    s = lax.axis_index("s")
    wid = c * NS + s

    pltpu.sync_copy(srcs_hbm.at[wid], src_v)
    pltpu.sync_copy(dsts_hbm.at[s], dst_v)
    if with_counts:
        pltpu.sync_copy(ones_hbm, ones_v)

    base = s * ROWS_PER_TILE
    pltpu.sync_copy(zrow_hbm, acc_s.at[pl.ds(base, ROWS_PER_TILE)])
    if with_counts:
        pltpu.sync_copy(zcnt_hbm, cacc_s.at[pl.ds(base, ROWS_PER_TILE)])
    plsc.subcore_barrier()

    def g_issue(j, b):
        pltpu.async_copy(x2_hbm.at[src_v.at[j]], bufs[b], gsems[b])

    def g_wait(j, b):
        pltpu.make_async_copy(x2_hbm.at[src_v.at[j]], bufs[b], gsems[b]).wait()

    def scat(j, b, parity):
        pltpu.sync_copy(bufs[b], acc_s.at[dst_v.at[j]], add=True)
        if with_counts:
            @pl.when(c == parity)
            def _():
                pltpu.sync_copy(ones_v, cacc_s.at[dst_v.at[j]], add=True)

    nb2 = 2 * NBUF
    for b in range(nb2):
        g_issue(b, b)

    def group(i, carry):
        k0 = nb2 * i
        for b in range(nb2):
            k = k0 + b
            g_wait(k, b)
            scat(k, b, k % 2)
            g_issue(k + nb2, b)
        return carry

    lax.fori_loop(0, NCHUNKS // nb2 - 1, group, 0)
    for b in range(nb2):
        k = NCHUNKS - nb2 + b
        g_wait(k, b)
        scat(k, b, k % 2)
    plsc.subcore_barrier()

    pltpu.sync_copy(acc_s.at[pl.ds(base, ROWS_PER_TILE)],
                    ssum_hbm.at[c, pl.ds(base, ROWS_PER_TILE)])
    if with_counts:
        pltpu.sync_copy(cacc_s.at[pl.ds(base, ROWS_PER_TILE)],
                        cnt_hbm.at[c, pl.ds(base, ROWS_PER_TILE)])


def _make_segment_sum_sc(with_counts):
    mesh = plsc.VectorSubcoreMesh(core_axis_name="c", subcore_axis_name="s")
    nbuf2 = 2 * NBUF

    def body(x2_hbm, srcs_hbm, dsts_hbm, zrow_hbm, zcnt_hbm, ones_hbm,
             *rest):
        if with_counts:
            outs, scratch = rest[:2], rest[2:]
        else:
            outs, scratch = rest[:1], rest[1:]
        ssum_hbm = outs[0]
        cnt_hbm = outs[1] if with_counts else None
        src_v, dst_v = scratch[:2]
        bufs = scratch[2:2 + nbuf2]
        idx = 2 + nbuf2
        if with_counts:
            ones_v, acc_s, cacc_s = scratch[idx], scratch[idx + 1], scratch[idx + 2]
            idx += 3
        else:
            ones_v, cacc_s = None, None
            acc_s = scratch[idx]
            idx += 1
        gsems = scratch[idx:idx + nbuf2]
        _sc_body(with_counts, x2_hbm, srcs_hbm, dsts_hbm, zrow_hbm, zcnt_hbm,
                 ones_hbm, ssum_hbm, cnt_hbm, src_v, dst_v,
                 bufs, ones_v, acc_s, cacc_s, gsems)

    out_type = [jax.ShapeDtypeStruct((NC, N_PAD, HD), jnp.float32)]
    scratch = [
        pltpu.VMEM((NCHUNKS, CHUNK), jnp.int32),
        pltpu.VMEM((NCHUNKS, CHUNK), jnp.int32),
    ] + [pltpu.VMEM((CHUNK, HD), jnp.float32) for _ in range(nbuf2)]
    if with_counts:
        out_type.append(jax.ShapeDtypeStruct((NC, N_PAD, CW), jnp.float32))
        scratch += [
            pltpu.VMEM((CHUNK, CW), jnp.float32),
            pltpu.VMEM_SHARED((N_PAD, HD), jnp.float32),
            pltpu.VMEM_SHARED((N_PAD, CW), jnp.float32),
        ]
    else:
        scratch += [pltpu.VMEM_SHARED((N_PAD, HD), jnp.float32)]
    scratch += [pltpu.SemaphoreType.DMA for _ in range(nbuf2)]

    f = pl.kernel(
        body,
        mesh=mesh,
        compiler_params=pltpu.CompilerParams(use_tc_tiling_on_sc=False),
        out_type=tuple(out_type),
        scratch_types=scratch,
    )

    def run(x2, srcs, dsts):
        zrow = jnp.zeros((ROWS_PER_TILE, HD), jnp.float32)
        zcnt = jnp.zeros((ROWS_PER_TILE, CW), jnp.float32)
        ones = jnp.ones((CHUNK, CW), jnp.float32)
        return f(x2, srcs, dsts, zrow, zcnt, ones)

    return run


_segsum_with_counts = _make_segment_sum_sc(True)
_segsum_no_counts = _make_segment_sum_sc(False)


def _tc_body(add_res, ps_ref, cs_ref, x_ref, wl_ref, bl_ref, wr_ref,
             lnw_ref, lnb_ref, res_ref, out_ref):
    ssum = jnp.concatenate([ps_ref[0], ps_ref[1]], axis=1)[:N]
    cnt = (cs_ref[0] + cs_ref[1])[:N, 0:1]
    agg = ssum / jnp.maximum(cnt, 1.0)
    t = (jnp.dot(agg, wl_ref[...], preferred_element_type=jnp.float32)
         + bl_ref[...]
         + jnp.dot(x_ref[...], wr_ref[...], preferred_element_type=jnp.float32))
    xc = t - jnp.mean(t)
    sd = jnp.sqrt(jnp.mean(xc * xc))
    y = (xc / (sd + 1e-5)) * lnw_ref[...] + lnb_ref[...]
    y = jnp.maximum(y, 0.0)
    if add_res:
        y = y + res_ref[...]
    out_ref[...] = y


def _dense_layer_tc(ps, cs, x, WlT, bl, WrT, lnw, lnb, res, add_res):
    body = functools.partial(_tc_body, add_res)
    return pl.pallas_call(
        body,
        out_shape=jax.ShapeDtypeStruct((N, D), jnp.float32),
    )(ps, cs, x, WlT, bl.reshape(1, D), WrT, lnw.reshape(1, D),
      lnb.reshape(1, D), res)


def kernel(x, edge_index, Wl1, bl1, Wr1, ln1_w, ln1_b,
           Wl2, bl2, Wr2, ln2_w, ln2_b):
    pad = NS * E_PAD_T - E
    src = jnp.concatenate([edge_index[0], jnp.zeros((pad,), jnp.int32)])
    dst = jnp.concatenate([edge_index[1], jnp.full((pad,), N, jnp.int32)])
    src2 = (src * 2).reshape(1, NS, NCHUNKS, CHUNK)
    srcs = jnp.concatenate([src2, src2 + 1], axis=0).reshape(
        NC * NS, NCHUNKS, CHUNK)
    dsts = dst.reshape(NS, NCHUNKS, CHUNK)

    ps1, cs1 = _segsum_with_counts(x.reshape(NC * N, HD), srcs, dsts)
    h1 = _dense_layer_tc(ps1, cs1, x, Wl1.T, bl1, Wr1.T, ln1_w, ln1_b,
                         x, add_res=False)
    (ps2,) = _segsum_no_counts(h1.reshape(NC * N, HD), srcs, dsts)
    h2 = _dense_layer_tc(ps2, cs1, h1, Wl2.T, bl2, Wr2.T, ln2_w, ln2_b,
                         x, add_res=True)
    return (h2, edge_index)


# speedup vs baseline: 1.9299x; 1.9299x over previous
import functools

import jax
import jax.numpy as jnp
from jax import lax
from jax.experimental import pallas as pl
from jax.experimental.pallas import tpu as pltpu
from jax.experimental.pallas import tpu_sc as plsc

N = 10000
D = 128
E = 320000

NC = 2
NS = 16
HD = D // NC
CHUNK = 128
EDGES_PER_TILE = E // NS
NCHUNKS = -(-EDGES_PER_TILE // CHUNK)
E_PAD_T = NCHUNKS * CHUNK
N_PAD = 10112
ROWS_PER_TILE = N_PAD // NS
CW = 8


def _sc_body(x2_hbm, srcs_hbm, dsts_hbm, zrow_hbm, zcnt_hbm, ones_hbm,
             ssum_hbm, cnt_hbm,
             src_v, dst_v, rows0_v, rows1_v, ones_v, acc_s, cacc_s,
             sem0, sem1):
    c = lax.axis_index("c")
    s = lax.axis_index("s")
    wid = c * NS + s

    pltpu.sync_copy(srcs_hbm.at[wid], src_v)
    pltpu.sync_copy(dsts_hbm.at[s], dst_v)
    pltpu.sync_copy(ones_hbm, ones_v)

    base = s * ROWS_PER_TILE
    pltpu.sync_copy(zrow_hbm, acc_s.at[pl.ds(base, ROWS_PER_TILE)])
    pltpu.sync_copy(zcnt_hbm, cacc_s.at[pl.ds(base, ROWS_PER_TILE)])
    plsc.subcore_barrier()

    def gather(j, buf, sem):
        return pltpu.async_copy(x2_hbm.at[src_v.at[j]], buf, sem)

    def scat(j, buf, parity):
        pltpu.sync_copy(buf, acc_s.at[dst_v.at[j]], add=True)

        @pl.when(c == parity)
        def _():
            pltpu.sync_copy(ones_v, cacc_s.at[dst_v.at[j]], add=True)

    gather(0, rows0_v, sem0)

    def pair(i, carry):
        k = 2 * i
        gather(k + 1, rows1_v, sem1)
        pltpu.make_async_copy(x2_hbm.at[src_v.at[k]], rows0_v, sem0).wait()
        scat(k, rows0_v, 0)
        gather(k + 2, rows0_v, sem0)
        pltpu.make_async_copy(x2_hbm.at[src_v.at[k + 1]], rows1_v, sem1).wait()
        scat(k + 1, rows1_v, 1)
        return carry

    lax.fori_loop(0, (NCHUNKS - 1) // 2, pair, 0)
    last = NCHUNKS - 1
    pltpu.make_async_copy(x2_hbm.at[src_v.at[last]], rows0_v, sem0).wait()
    scat(last, rows0_v, 0)
    plsc.subcore_barrier()

    pltpu.sync_copy(acc_s.at[pl.ds(base, ROWS_PER_TILE)],
                    ssum_hbm.at[c, pl.ds(base, ROWS_PER_TILE)])
    pltpu.sync_copy(cacc_s.at[pl.ds(base, ROWS_PER_TILE)],
                    cnt_hbm.at[c, pl.ds(base, ROWS_PER_TILE)])


def _segment_sum_sc(x2, srcs, dsts):
    zrow = jnp.zeros((ROWS_PER_TILE, HD), jnp.float32)
    zcnt = jnp.zeros((ROWS_PER_TILE, CW), jnp.float32)
    ones = jnp.ones((CHUNK, CW), jnp.float32)
    mesh = plsc.VectorSubcoreMesh(core_axis_name="c", subcore_axis_name="s")
    f = pl.kernel(
        _sc_body,
        mesh=mesh,
        compiler_params=pltpu.CompilerParams(use_tc_tiling_on_sc=False),
        out_type=(
            jax.ShapeDtypeStruct((NC, N_PAD, HD), jnp.float32),
            jax.ShapeDtypeStruct((NC, N_PAD, CW), jnp.float32),
        ),
        scratch_types=[
            pltpu.VMEM((NCHUNKS, CHUNK), jnp.int32),
            pltpu.VMEM((NCHUNKS, CHUNK), jnp.int32),
            pltpu.VMEM((CHUNK, HD), jnp.float32),
            pltpu.VMEM((CHUNK, HD), jnp.float32),
            pltpu.VMEM((CHUNK, CW), jnp.float32),
            pltpu.VMEM_SHARED((N_PAD, HD), jnp.float32),
            pltpu.VMEM_SHARED((N_PAD, CW), jnp.float32),
            pltpu.SemaphoreType.DMA,
            pltpu.SemaphoreType.DMA,
        ],
    )
    return f(x2, srcs, dsts, zrow, zcnt, ones)


def _tc_body(add_res, ps_ref, cs_ref, x_ref, wl_ref, bl_ref, wr_ref,
             lnw_ref, lnb_ref, res_ref, out_ref):
    ssum = jnp.concatenate([ps_ref[0], ps_ref[1]], axis=1)[:N]
    cnt = (cs_ref[0] + cs_ref[1])[:N, 0:1]
    agg = ssum / jnp.maximum(cnt, 1.0)
    t = (jnp.dot(agg, wl_ref[...], preferred_element_type=jnp.float32)
         + bl_ref[...]
         + jnp.dot(x_ref[...], wr_ref[...], preferred_element_type=jnp.float32))
    xc = t - jnp.mean(t)
    sd = jnp.sqrt(jnp.mean(xc * xc))
    y = (xc / (sd + 1e-5)) * lnw_ref[...] + lnb_ref[...]
    y = jnp.maximum(y, 0.0)
    if add_res:
        y = y + res_ref[...]
    out_ref[...] = y


def _dense_layer_tc(ps, cs, x, WlT, bl, WrT, lnw, lnb, res, add_res):
    body = functools.partial(_tc_body, add_res)
    return pl.pallas_call(
        body,
        out_shape=jax.ShapeDtypeStruct((N, D), jnp.float32),
    )(ps, cs, x, WlT, bl.reshape(1, D), WrT, lnw.reshape(1, D),
      lnb.reshape(1, D), res)


def kernel(x, edge_index, Wl1, bl1, Wr1, ln1_w, ln1_b,
           Wl2, bl2, Wr2, ln2_w, ln2_b):
    pad = NS * E_PAD_T - E
    src = jnp.concatenate([edge_index[0], jnp.zeros((pad,), jnp.int32)])
    dst = jnp.concatenate([edge_index[1], jnp.full((pad,), N, jnp.int32)])
    src2 = (src * 2).reshape(1, NS, NCHUNKS, CHUNK)
    srcs = jnp.concatenate([src2, src2 + 1], axis=0).reshape(
        NC * NS, NCHUNKS, CHUNK)
    dsts = dst.reshape(NS, NCHUNKS, CHUNK)

    ps1, cs1 = _segment_sum_sc(x.reshape(NC * N, HD), srcs, dsts)
    h1 = _dense_layer_tc(ps1, cs1, x, Wl1.T, bl1, Wr1.T, ln1_w, ln1_b,
                         x, add_res=False)
    ps2, cs2 = _segment_sum_sc(h1.reshape(NC * N, HD), srcs, dsts)
    h2 = _dense_layer_tc(ps2, cs2, h1, Wl2.T, bl2, Wr2.T, ln2_w, ln2_b,
                         x, add_res=True)
    return (h2, edge_index)


# speedup vs baseline: 2.2185x; 1.1495x over previous
import functools

import jax
import jax.numpy as jnp
from jax import lax
from jax.experimental import pallas as pl
from jax.experimental.pallas import tpu as pltpu
from jax.experimental.pallas import tpu_sc as plsc

N = 10000
D = 128
E = 320000

NC = 2
NS = 16
HD = D // NC
CHUNK = 128
EDGES_PER_TILE = E // NS
NCHUNKS = -(-EDGES_PER_TILE // CHUNK)
E_PAD_T = NCHUNKS * CHUNK
N_PAD = 10112
ROWS_PER_TILE = N_PAD // NS
CW = 8


def _sc_body(x2_hbm, srcs_hbm, dsts_hbm, zrow_hbm, zcnt_hbm, ones_hbm,
             ssum_hbm, cnt_hbm,
             src_v, dst_v, rows0_v, rows1_v, rows2_v, rows3_v, ones_v,
             acc_s, cacc_s, sem0, sem1, sem2, sem3):
    c = lax.axis_index("c")
    s = lax.axis_index("s")
    wid = c * NS + s

    pltpu.sync_copy(srcs_hbm.at[wid], src_v)
    pltpu.sync_copy(dsts_hbm.at[s], dst_v)
    pltpu.sync_copy(ones_hbm, ones_v)

    base = s * ROWS_PER_TILE
    pltpu.sync_copy(zrow_hbm, acc_s.at[pl.ds(base, ROWS_PER_TILE)])
    pltpu.sync_copy(zcnt_hbm, cacc_s.at[pl.ds(base, ROWS_PER_TILE)])
    plsc.subcore_barrier()

    def gather(j, buf, sem):
        return pltpu.async_copy(x2_hbm.at[src_v.at[j]], buf, sem)

    def scat(j, buf, parity):
        pltpu.sync_copy(buf, acc_s.at[dst_v.at[j]], add=True)

        @pl.when(c == parity)
        def _():
            pltpu.sync_copy(ones_v, cacc_s.at[dst_v.at[j]], add=True)

    bufs = (rows0_v, rows1_v, rows2_v, rows3_v)
    sems = (sem0, sem1, sem2, sem3)

    def g_wait(j, b):
        pltpu.make_async_copy(x2_hbm.at[src_v.at[j]], bufs[b], sems[b]).wait()

    for b in range(4):
        gather(b, bufs[b], sems[b])

    def quad(i, carry):
        k0 = 4 * i
        for b in range(4):
            k = k0 + b
            g_wait(k, b)
            scat(k, bufs[b], b % 2)
            gather(k + 4, bufs[b], sems[b])
        return carry

    lax.fori_loop(0, NCHUNKS // 4 - 1, quad, 0)
    for b in range(4):
        k = NCHUNKS - 5 + b
        g_wait(k, b)
        scat(k, bufs[b], k % 2)
    gather(NCHUNKS - 1, rows0_v, sem0)
    g_wait(NCHUNKS - 1, 0)
    scat(NCHUNKS - 1, rows0_v, (NCHUNKS - 1) % 2)
    plsc.subcore_barrier()

    pltpu.sync_copy(acc_s.at[pl.ds(base, ROWS_PER_TILE)],
                    ssum_hbm.at[c, pl.ds(base, ROWS_PER_TILE)])
    pltpu.sync_copy(cacc_s.at[pl.ds(base, ROWS_PER_TILE)],
                    cnt_hbm.at[c, pl.ds(base, ROWS_PER_TILE)])


def _segment_sum_sc(x2, srcs, dsts):
    zrow = jnp.zeros((ROWS_PER_TILE, HD), jnp.float32)
    zcnt = jnp.zeros((ROWS_PER_TILE, CW), jnp.float32)
    ones = jnp.ones((CHUNK, CW), jnp.float32)
    mesh = plsc.VectorSubcoreMesh(core_axis_name="c", subcore_axis_name="s")
    f = pl.kernel(
        _sc_body,
        mesh=mesh,
        compiler_params=pltpu.CompilerParams(use_tc_tiling_on_sc=False),
        out_type=(
            jax.ShapeDtypeStruct((NC, N_PAD, HD), jnp.float32),
            jax.ShapeDtypeStruct((NC, N_PAD, CW), jnp.float32),
        ),
        scratch_types=[
            pltpu.VMEM((NCHUNKS, CHUNK), jnp.int32),
            pltpu.VMEM((NCHUNKS, CHUNK), jnp.int32),
            pltpu.VMEM((CHUNK, HD), jnp.float32),
            pltpu.VMEM((CHUNK, HD), jnp.float32),
            pltpu.VMEM((CHUNK, HD), jnp.float32),
            pltpu.VMEM((CHUNK, HD), jnp.float32),
            pltpu.VMEM((CHUNK, CW), jnp.float32),
            pltpu.VMEM_SHARED((N_PAD, HD), jnp.float32),
            pltpu.VMEM_SHARED((N_PAD, CW), jnp.float32),
            pltpu.SemaphoreType.DMA,
            pltpu.SemaphoreType.DMA,
            pltpu.SemaphoreType.DMA,
            pltpu.SemaphoreType.DMA,
        ],
    )
    return f(x2, srcs, dsts, zrow, zcnt, ones)


def _tc_body(add_res, ps_ref, cs_ref, x_ref, wl_ref, bl_ref, wr_ref,
             lnw_ref, lnb_ref, res_ref, out_ref):
    ssum = jnp.concatenate([ps_ref[0], ps_ref[1]], axis=1)[:N]
    cnt = (cs_ref[0] + cs_ref[1])[:N, 0:1]
    agg = ssum / jnp.maximum(cnt, 1.0)
    t = (jnp.dot(agg, wl_ref[...], preferred_element_type=jnp.float32)
         + bl_ref[...]
         + jnp.dot(x_ref[...], wr_ref[...], preferred_element_type=jnp.float32))
    xc = t - jnp.mean(t)
    sd = jnp.sqrt(jnp.mean(xc * xc))
    y = (xc / (sd + 1e-5)) * lnw_ref[...] + lnb_ref[...]
    y = jnp.maximum(y, 0.0)
    if add_res:
        y = y + res_ref[...]
    out_ref[...] = y


def _dense_layer_tc(ps, cs, x, WlT, bl, WrT, lnw, lnb, res, add_res):
    body = functools.partial(_tc_body, add_res)
    return pl.pallas_call(
        body,
        out_shape=jax.ShapeDtypeStruct((N, D), jnp.float32),
    )(ps, cs, x, WlT, bl.reshape(1, D), WrT, lnw.reshape(1, D),
      lnb.reshape(1, D), res)


def kernel(x, edge_index, Wl1, bl1, Wr1, ln1_w, ln1_b,
           Wl2, bl2, Wr2, ln2_w, ln2_b):
    pad = NS * E_PAD_T - E
    src = jnp.concatenate([edge_index[0], jnp.zeros((pad,), jnp.int32)])
    dst = jnp.concatenate([edge_index[1], jnp.full((pad,), N, jnp.int32)])
    src2 = (src * 2).reshape(1, NS, NCHUNKS, CHUNK)
    srcs = jnp.concatenate([src2, src2 + 1], axis=0).reshape(
        NC * NS, NCHUNKS, CHUNK)
    dsts = dst.reshape(NS, NCHUNKS, CHUNK)

    ps1, cs1 = _segment_sum_sc(x.reshape(NC * N, HD), srcs, dsts)
    h1 = _dense_layer_tc(ps1, cs1, x, Wl1.T, bl1, Wr1.T, ln1_w, ln1_b,
                         x, add_res=False)
    ps2, cs2 = _segment_sum_sc(h1.reshape(NC * N, HD), srcs, dsts)
    h2 = _dense_layer_tc(ps2, cs2, h1, Wl2.T, bl2, Wr2.T, ln2_w, ln2_b,
                         x, add_res=True)
    return (h2, edge_index)


# speedup vs baseline: 2.2851x; 1.0300x over previous
import functools

import jax
import jax.numpy as jnp
from jax import lax
from jax.experimental import pallas as pl
from jax.experimental.pallas import tpu as pltpu
from jax.experimental.pallas import tpu_sc as plsc

N = 10000
D = 128
E = 320000

NC = 2
NS = 16
HD = D // NC
CHUNK = 128
EDGES_PER_TILE = E // NS
NCHUNKS = -(-EDGES_PER_TILE // CHUNK)
E_PAD_T = NCHUNKS * CHUNK
N_PAD = 10112
ROWS_PER_TILE = N_PAD // NS
CW = 8


def _sc_body(with_counts, x2_hbm, srcs_hbm, dsts_hbm, zrow_hbm, zcnt_hbm,
             ones_hbm, ssum_hbm, cnt_hbm,
             src_v, dst_v, rows0_v, rows1_v, rows2_v, rows3_v, ones_v,
             acc_s, cacc_s, sem0, sem1, sem2, sem3):
    c = lax.axis_index("c")
    s = lax.axis_index("s")
    wid = c * NS + s

    pltpu.sync_copy(srcs_hbm.at[wid], src_v)
    pltpu.sync_copy(dsts_hbm.at[s], dst_v)
    if with_counts:
        pltpu.sync_copy(ones_hbm, ones_v)

    base = s * ROWS_PER_TILE
    pltpu.sync_copy(zrow_hbm, acc_s.at[pl.ds(base, ROWS_PER_TILE)])
    if with_counts:
        pltpu.sync_copy(zcnt_hbm, cacc_s.at[pl.ds(base, ROWS_PER_TILE)])
    plsc.subcore_barrier()

    def gather(j, buf, sem):
        return pltpu.async_copy(x2_hbm.at[src_v.at[j]], buf, sem)

    def scat(j, buf, parity):
        pltpu.sync_copy(buf, acc_s.at[dst_v.at[j]], add=True)

        if with_counts:
            @pl.when(c == parity)
            def _():
                pltpu.sync_copy(ones_v, cacc_s.at[dst_v.at[j]], add=True)

    bufs = (rows0_v, rows1_v, rows2_v, rows3_v)
    sems = (sem0, sem1, sem2, sem3)

    def g_wait(j, b):
        pltpu.make_async_copy(x2_hbm.at[src_v.at[j]], bufs[b], sems[b]).wait()

    for b in range(4):
        gather(b, bufs[b], sems[b])

    def quad(i, carry):
        k0 = 4 * i
        for b in range(4):
            k = k0 + b
            g_wait(k, b)
            scat(k, bufs[b], b % 2)
            gather(k + 4, bufs[b], sems[b])
        return carry

    lax.fori_loop(0, NCHUNKS // 4 - 1, quad, 0)
    for b in range(4):
        k = NCHUNKS - 5 + b
        g_wait(k, b)
        scat(k, bufs[b], k % 2)
    gather(NCHUNKS - 1, rows0_v, sem0)
    g_wait(NCHUNKS - 1, 0)
    scat(NCHUNKS - 1, rows0_v, (NCHUNKS - 1) % 2)
    plsc.subcore_barrier()

    pltpu.sync_copy(acc_s.at[pl.ds(base, ROWS_PER_TILE)],
                    ssum_hbm.at[c, pl.ds(base, ROWS_PER_TILE)])
    if with_counts:
        pltpu.sync_copy(cacc_s.at[pl.ds(base, ROWS_PER_TILE)],
                        cnt_hbm.at[c, pl.ds(base, ROWS_PER_TILE)])


def _segment_sum_sc(x2, srcs, dsts, with_counts=True):
    zrow = jnp.zeros((ROWS_PER_TILE, HD), jnp.float32)
    zcnt = jnp.zeros((ROWS_PER_TILE, CW), jnp.float32)
    ones = jnp.ones((CHUNK, CW), jnp.float32)
    mesh = plsc.VectorSubcoreMesh(core_axis_name="c", subcore_axis_name="s")
    f = pl.kernel(
        functools.partial(_sc_body, with_counts),
        mesh=mesh,
        compiler_params=pltpu.CompilerParams(use_tc_tiling_on_sc=False),
        out_type=(
            jax.ShapeDtypeStruct((NC, N_PAD, HD), jnp.float32),
            jax.ShapeDtypeStruct((NC, N_PAD, CW), jnp.float32),
        ),
        scratch_types=[
            pltpu.VMEM((NCHUNKS, CHUNK), jnp.int32),
            pltpu.VMEM((NCHUNKS, CHUNK), jnp.int32),
            pltpu.VMEM((CHUNK, HD), jnp.float32),
            pltpu.VMEM((CHUNK, HD), jnp.float32),
            pltpu.VMEM((CHUNK, HD), jnp.float32),
            pltpu.VMEM((CHUNK, HD), jnp.float32),
            pltpu.VMEM((CHUNK, CW), jnp.float32),
            pltpu.VMEM_SHARED((N_PAD, HD), jnp.float32),
            pltpu.VMEM_SHARED((N_PAD, CW), jnp.float32),
            pltpu.SemaphoreType.DMA,
            pltpu.SemaphoreType.DMA,
            pltpu.SemaphoreType.DMA,
            pltpu.SemaphoreType.DMA,
        ],
    )
    return f(x2, srcs, dsts, zrow, zcnt, ones)


def _tc_body(add_res, ps_ref, cs_ref, x_ref, wl_ref, bl_ref, wr_ref,
             lnw_ref, lnb_ref, res_ref, out_ref):
    ssum = jnp.concatenate([ps_ref[0], ps_ref[1]], axis=1)[:N]
    cnt = (cs_ref[0] + cs_ref[1])[:N, 0:1]
    agg = ssum / jnp.maximum(cnt, 1.0)
    t = (jnp.dot(agg, wl_ref[...], preferred_element_type=jnp.float32)
         + bl_ref[...]
         + jnp.dot(x_ref[...], wr_ref[...], preferred_element_type=jnp.float32))
    xc = t - jnp.mean(t)
    sd = jnp.sqrt(jnp.mean(xc * xc))
    y = (xc / (sd + 1e-5)) * lnw_ref[...] + lnb_ref[...]
    y = jnp.maximum(y, 0.0)
    if add_res:
        y = y + res_ref[...]
    out_ref[...] = y


def _dense_layer_tc(ps, cs, x, WlT, bl, WrT, lnw, lnb, res, add_res):
    body = functools.partial(_tc_body, add_res)
    return pl.pallas_call(
        body,
        out_shape=jax.ShapeDtypeStruct((N, D), jnp.float32),
    )(ps, cs, x, WlT, bl.reshape(1, D), WrT, lnw.reshape(1, D),
      lnb.reshape(1, D), res)


def kernel(x, edge_index, Wl1, bl1, Wr1, ln1_w, ln1_b,
           Wl2, bl2, Wr2, ln2_w, ln2_b):
    pad = NS * E_PAD_T - E
    src = jnp.concatenate([edge_index[0], jnp.zeros((pad,), jnp.int32)])
    dst = jnp.concatenate([edge_index[1], jnp.full((pad,), N, jnp.int32)])
    src2 = (src * 2).reshape(1, NS, NCHUNKS, CHUNK)
    srcs = jnp.concatenate([src2, src2 + 1], axis=0).reshape(
        NC * NS, NCHUNKS, CHUNK)
    dsts = dst.reshape(NS, NCHUNKS, CHUNK)

    ps1, cs1 = _segment_sum_sc(x.reshape(NC * N, HD), srcs, dsts)
    h1 = _dense_layer_tc(ps1, cs1, x, Wl1.T, bl1, Wr1.T, ln1_w, ln1_b,
                         x, add_res=False)
    ps2, _ = _segment_sum_sc(h1.reshape(NC * N, HD), srcs, dsts,
                             with_counts=False)
    h2 = _dense_layer_tc(ps2, cs1, h1, Wl2.T, bl2, Wr2.T, ln2_w, ln2_b,
                         x, add_res=True)
    return (h2, edge_index)
